# Initial kernel scaffold; baseline (speedup 1.0000x reference)
import jax
import jax.numpy as jnp
from jax.experimental import pallas as pl


def kernel(user_emb, entity_emb, latent_emb, relation_emb, disen_weight_att, ent_rel_w, usr_cls_w, edge_index, edge_type, edge_imp, inter_rows, inter_cols, inter_vals, cls_rows, cls_cols, cls_vals):
    raise NotImplementedError("write your pallas kernel here")


# speedup vs baseline: 1.4626x; 1.4626x over previous
import functools

import jax
import jax.numpy as jnp
from jax import lax
from jax.experimental import pallas as pl
from jax.experimental.pallas import tpu as pltpu
from jax.experimental.pallas import tpu_sc as plsc

N_ENT = 10000
N_USERS = 4096
N_ITEMS = 5000
N_REL = 24
N_FACTORS = 4
N_CLUSTERS = 4
D = 128
N_EDGES = 320000
NNZ = 200000
N_HOPS = 2

NC = 2
NS = 16
NW = NC * NS
SB = 128

E_PAD = 327680
EPT = E_PAD // NW
NNZ_PAD = 212992
ZPT = NNZ_PAD // NW

ENT_STRIPE = 632
ENT_LAST = N_ENT - 15 * ENT_STRIPE
USR_STRIPE = N_USERS // NS


def _pad1(x, n, val):
    return jnp.pad(x, (0, n - x.shape[0]), constant_values=val)


def _softmax_body(ncols, x_ref, w_ref, o_ref):
    z = lax.dot_general(x_ref[...], w_ref[...], (((1,), (1,)), ((), ())),
                        preferred_element_type=jnp.float32)
    col = lax.broadcasted_iota(jnp.int32, z.shape, 1)
    z = jnp.where(col < ncols, z, -jnp.inf)
    z = z - jnp.max(z, axis=1, keepdims=True)
    e = jnp.exp(z)
    o_ref[...] = e / jnp.sum(e, axis=1, keepdims=True)


def _prep_softmax(x, w_pad, ncols, blk):
    n, _ = x.shape
    wp = w_pad.shape[0]
    return pl.pallas_call(
        functools.partial(_softmax_body, ncols),
        grid=(n // blk,),
        in_specs=[pl.BlockSpec((blk, D), lambda i: (i, 0)),
                  pl.BlockSpec((wp, D), lambda i: (0, 0))],
        out_specs=pl.BlockSpec((blk, wp), lambda i: (i, 0)),
        out_shape=jax.ShapeDtypeStruct((n, wp), jnp.float32),
    )(x, w_pad)


def _finish_body(parts_ref, res_ref, e_ref, r_ref):
    x = parts_ref[0] + parts_ref[1]
    nrm = jnp.sqrt(jnp.sum(x * x, axis=1, keepdims=True))
    y = x / jnp.maximum(nrm, 1e-12)
    e_ref[...] = y
    r_ref[...] = res_ref[...] + y


def _finish(parts, res, blk):
    n = res.shape[0]
    return pl.pallas_call(
        _finish_body,
        grid=(n // blk,),
        in_specs=[pl.BlockSpec((2, blk, D), lambda i: (0, i, 0)),
                  pl.BlockSpec((blk, D), lambda i: (i, 0))],
        out_specs=[pl.BlockSpec((blk, D), lambda i: (i, 0)),
                   pl.BlockSpec((blk, D), lambda i: (i, 0))],
        out_shape=[jax.ShapeDtypeStruct((n, D), jnp.float32),
                   jax.ShapeDtypeStruct((n, D), jnp.float32)],
    )(parts, res)


def _cor_body(w_ref, o_ref):
    f = N_FACTORS
    n = N_REL
    ri = lax.broadcasted_iota(jnp.int32, (n, n), 0)
    ci = lax.broadcasted_iota(jnp.int32, (n, n), 1)

    def centered_dist(row):
        x = jnp.broadcast_to(row, (n, n))
        dg = jnp.where(ri == ci, x, 0.0)
        vcol = jnp.sum(dg, axis=1, keepdims=True)
        outer2 = (vcol * row) * 2.0
        sq = vcol * vcol
        sqr = row * row
        a = jnp.sqrt(jnp.maximum(sq - outer2 + sqr, 0.0) + 1e-08)
        return (a - jnp.mean(a, axis=0, keepdims=True)
                - jnp.mean(a, axis=1, keepdims=True) + jnp.mean(a))

    mats = [centered_dist(w_ref[i:i + 1, :]) for i in range(f)]
    n2 = float(n * n)
    cor = jnp.float32(0.0)
    for i in range(f):
        for j in range(i + 1, f):
            ab = jnp.sqrt(jnp.maximum(jnp.sum(mats[i] * mats[j]) / n2, 0.0) + 1e-08)
            aa = jnp.sqrt(jnp.maximum(jnp.sum(mats[i] * mats[i]) / n2, 0.0) + 1e-08)
            bb = jnp.sqrt(jnp.maximum(jnp.sum(mats[j] * mats[j]) / n2, 0.0) + 1e-08)
            cor = cor + ab / jnp.sqrt(aa * bb + 1e-08)
    o_ref[...] = jnp.broadcast_to(cor, (1, 1))


def _cor(disen_weight_att):
    out = pl.pallas_call(
        _cor_body,
        out_shape=jax.ShapeDtypeStruct((1, 1), jnp.float32),
    )(disen_weight_att)
    return out[0, 0]


KE = 256
NSB_E = KE // SB
NCH_E = EPT // KE
KU = 512
NSB_U = KU // SB
NCH_U = ZPT // KU


def _zero_buf(buf, k):
    def _row(r, _):
        for c in range(8):
            buf[r, pl.ds(c * 16, 16)] = jnp.zeros((16,), jnp.float32)
        return _
    lax.fori_loop(0, k, _row, None)


def _zero_stripe(buf, k, acc, off, rows):
    done = 0
    while done < rows:
        n = min(k, rows - done)
        pltpu.sync_copy(buf.at[pl.ds(0, n)], acc.at[pl.ds(off + done, n)])
        done += n


def _sc_edge_body(ee, attp, rel,
                  tail1, head1, etyp, eimp, eatt,
                  out_e,
                  ee_t, asel_v, rel_v, typ_v, w_v, idx_a, idx_b, idx_c,
                  ent_acc):
    cid = lax.axis_index("c")
    sid = lax.axis_index("s")
    wid = sid * NC + cid

    pltpu.sync_copy(rel, rel_v)
    _zero_buf(ee_t, KE)

    @pl.when(sid < 15)
    def _():
        _zero_stripe(ee_t, KE, ent_acc, sid * ENT_STRIPE, ENT_STRIPE)

    @pl.when(sid == 15)
    def _():
        _zero_stripe(ee_t, KE, ent_acc, 15 * ENT_STRIPE, ENT_LAST)

    plsc.subcore_barrier()

    def edge_chunk(ck, _):
        base = wid * EPT + ck * KE
        pltpu.sync_copy(etyp.at[pl.ds(base, KE)], typ_v)
        pltpu.sync_copy(eimp.at[pl.ds(base, KE)], w_v)
        for j in range(NSB_E):
            pltpu.sync_copy(tail1.at[pl.ds(base + j * SB, SB)], idx_a.at[j])
            pltpu.sync_copy(head1.at[pl.ds(base + j * SB, SB)], idx_b.at[j])
            pltpu.sync_copy(eatt.at[pl.ds(base + j * SB, SB)], idx_c.at[j])
        for j in range(NSB_E):
            pltpu.sync_copy(ee.at[idx_a.at[j]], ee_t.at[pl.ds(j * SB, SB)])
            pltpu.sync_copy(attp.at[idx_c.at[j]], asel_v.at[pl.ds(j * SB, SB)])

        def group(g, _):
            t16 = typ_v[pl.ds(g * 16, 16)]
            wful = asel_v[pl.ds(g * 16, 16)] * w_v[pl.ds(g * 16, 16)]
            for j in range(16):
                w = wful[j]
                t = t16[j]
                e = g * 16 + j
                for c in range(8):
                    sl = pl.ds(c * 16, 16)
                    ee_t[e, sl] = ee_t[e, sl] * rel_v[t, sl] * w
            return _
        lax.fori_loop(0, KE // 16, group, None)
        for j in range(NSB_E):
            pltpu.sync_copy(ee_t.at[pl.ds(j * SB, SB)],
                            ent_acc.at[idx_b.at[j]], add=True)
        return _
    lax.fori_loop(0, NCH_E, edge_chunk, None)

    plsc.subcore_barrier()

    @pl.when(sid < 15)
    def _():
        pltpu.sync_copy(ent_acc.at[pl.ds(sid * ENT_STRIPE, ENT_STRIPE)],
                        out_e.at[cid, pl.ds(sid * ENT_STRIPE, ENT_STRIPE)])

    @pl.when(sid == 15)
    def _():
        pltpu.sync_copy(ent_acc.at[pl.ds(15 * ENT_STRIPE, ENT_LAST)],
                        out_e.at[cid, pl.ds(15 * ENT_STRIPE, ENT_LAST)])


def _sc_user_body(ee, uclsp, rel,
                  icol1, irow1, ival,
                  ccol1, cu1, cuc, cval,
                  out_u,
                  ee_t, asel_v, rel_v, rs_v, w_v, idx_a, idx_b, idx_c,
                  usr_acc):
    cid = lax.axis_index("c")
    sid = lax.axis_index("s")
    wid = sid * NC + cid

    pltpu.sync_copy(rel, rel_v)
    for c in range(8):
        sl = pl.ds(c * 16, 16)
        acc = jnp.zeros((16,), jnp.float32)
        for r in range(N_REL):
            acc = acc + rel_v[r, sl]
        rs_v[sl] = acc

    _zero_buf(ee_t, KU)
    _zero_stripe(ee_t, KU, usr_acc, sid * USR_STRIPE, USR_STRIPE)
    plsc.subcore_barrier()

    def inter_chunk(ck, _):
        base = wid * ZPT + ck * KU
        pltpu.sync_copy(ival.at[pl.ds(base, KU)], w_v)
        for j in range(NSB_U):
            pltpu.sync_copy(icol1.at[pl.ds(base + j * SB, SB)], idx_a.at[j])
            pltpu.sync_copy(irow1.at[pl.ds(base + j * SB, SB)], idx_b.at[j])
        for j in range(NSB_U):
            pltpu.sync_copy(ee.at[idx_a.at[j]], ee_t.at[pl.ds(j * SB, SB)])

        def group(g, _):
            wful = w_v[pl.ds(g * 16, 16)]
            for j in range(16):
                w = wful[j]
                e = g * 16 + j
                for c in range(8):
                    sl = pl.ds(c * 16, 16)
                    ee_t[e, sl] = ee_t[e, sl] * w
            return _
        lax.fori_loop(0, KU // 16, group, None)
        for j in range(NSB_U):
            pltpu.sync_copy(ee_t.at[pl.ds(j * SB, SB)],
                            usr_acc.at[idx_b.at[j]], add=True)
        return _
    lax.fori_loop(0, NCH_U, inter_chunk, None)

    def cls_chunk(ck, _):
        base = wid * ZPT + ck * KU
        pltpu.sync_copy(cval.at[pl.ds(base, KU)], w_v)
        for j in range(NSB_U):
            pltpu.sync_copy(ccol1.at[pl.ds(base + j * SB, SB)], idx_a.at[j])
            pltpu.sync_copy(cu1.at[pl.ds(base + j * SB, SB)], idx_b.at[j])
            pltpu.sync_copy(cuc.at[pl.ds(base + j * SB, SB)], idx_c.at[j])
        for j in range(NSB_U):
            pltpu.sync_copy(ee.at[idx_a.at[j]], ee_t.at[pl.ds(j * SB, SB)])
            pltpu.sync_copy(uclsp.at[idx_c.at[j]], asel_v.at[pl.ds(j * SB, SB)])

        def group(g, _):
            wful = asel_v[pl.ds(g * 16, 16)] * w_v[pl.ds(g * 16, 16)]
            for j in range(16):
                w = wful[j]
                e = g * 16 + j
                for c in range(8):
                    sl = pl.ds(c * 16, 16)
                    ee_t[e, sl] = ee_t[e, sl] * rs_v[sl] * w
            return _
        lax.fori_loop(0, KU // 16, group, None)
        for j in range(NSB_U):
            pltpu.sync_copy(ee_t.at[pl.ds(j * SB, SB)],
                            usr_acc.at[idx_b.at[j]], add=True)
        return _
    lax.fori_loop(0, NCH_U, cls_chunk, None)

    plsc.subcore_barrier()
    pltpu.sync_copy(usr_acc.at[pl.ds(sid * USR_STRIPE, USR_STRIPE)],
                    out_u.at[cid, pl.ds(sid * USR_STRIPE, USR_STRIPE)])


def _sc_edge(ee, attp, rel, edge_in):
    tail1, head1, etyp, eimp, eatt = edge_in
    mesh = plsc.VectorSubcoreMesh(core_axis_name="c", subcore_axis_name="s")
    f32 = jnp.float32
    kfn = pl.kernel(
        _sc_edge_body,
        mesh=mesh,
        out_type=jax.ShapeDtypeStruct((NC, N_ENT, D), f32),
        scratch_types=[
            pltpu.VMEM((KE, D), f32),
            pltpu.VMEM((KE,), f32),
            pltpu.VMEM((N_REL, D), f32),
            pltpu.VMEM((KE,), jnp.int32),
            pltpu.VMEM((KE,), f32),
            pltpu.VMEM((NSB_E, SB), jnp.int32),
            pltpu.VMEM((NSB_E, SB), jnp.int32),
            pltpu.VMEM((NSB_E, SB), jnp.int32),
            pltpu.VMEM_SHARED((N_ENT, D), f32),
        ],
    )
    return kfn(ee, attp, rel, tail1, head1, etyp, eimp, eatt)


def _sc_user(ee, uclsp, rel, inter_in, cls_in):
    icol1, irow1, ival = inter_in
    ccol1, cu1, cuc, cval = cls_in
    mesh = plsc.VectorSubcoreMesh(core_axis_name="c", subcore_axis_name="s")
    f32 = jnp.float32
    kfn = pl.kernel(
        _sc_user_body,
        mesh=mesh,
        out_type=jax.ShapeDtypeStruct((NC, N_USERS, D), f32),
        scratch_types=[
            pltpu.VMEM((KU, D), f32),
            pltpu.VMEM((KU,), f32),
            pltpu.VMEM((N_REL, D), f32),
            pltpu.VMEM((D,), f32),
            pltpu.VMEM((KU,), f32),
            pltpu.VMEM((NSB_U, SB), jnp.int32),
            pltpu.VMEM((NSB_U, SB), jnp.int32),
            pltpu.VMEM((NSB_U, SB), jnp.int32),
            pltpu.VMEM_SHARED((N_USERS, D), f32),
        ],
    )
    return kfn(ee, uclsp, rel, icol1, irow1, ival, ccol1, cu1, cuc, cval)


def kernel(user_emb, entity_emb, latent_emb, relation_emb, disen_weight_att,
           ent_rel_w, usr_cls_w, edge_index, edge_type, edge_imp,
           inter_rows, inter_cols, inter_vals, cls_rows, cls_cols, cls_vals):
    del latent_emb, ent_rel_w
    i32 = jnp.int32
    head = edge_index[0].astype(i32)
    tail = edge_index[1].astype(i32)

    tail2 = _pad1(tail, E_PAD, 0)
    head2 = _pad1(head, E_PAD, 0)
    etyp = _pad1(edge_type.astype(i32), E_PAD, 0)
    eimp = _pad1(edge_imp, E_PAD, 0.0)

    icol2 = _pad1(inter_cols.astype(i32), NNZ_PAD, 0)
    irow2 = _pad1(inter_rows.astype(i32), NNZ_PAD, 0)
    ival = _pad1(inter_vals, NNZ_PAD, 0.0)

    cr = cls_rows.astype(i32)
    ccol2 = _pad1(cls_cols.astype(i32), NNZ_PAD, 0)
    cu2 = _pad1(cr % N_USERS, NNZ_PAD, 0)
    cuc = _pad1((cr % N_USERS) * 16 + cr // N_USERS, NNZ_PAD, 0)
    cval = _pad1(cls_vals, NNZ_PAD, 0.0)
    eatt = _pad1(head * 32 + edge_type.astype(i32), E_PAD, 0)

    rel_p = jnp.pad(relation_emb, ((0, 32 - N_REL), (0, 0)))
    uclsw_p = jnp.pad(usr_cls_w, ((0, 16 - N_CLUSTERS), (0, 0)))

    ee = entity_emb
    ue = user_emb
    ent_res = entity_emb
    usr_res = user_emb
    for _ in range(N_HOPS):
        attp = _prep_softmax(ee, rel_p, N_REL, 1000)
        uclsp = _prep_softmax(ue, uclsw_p, N_CLUSTERS, 1024)
        ent_parts = _sc_edge(ee, attp.reshape(-1), relation_emb,
                             (tail2, head2, etyp, eimp, eatt))
        usr_parts = _sc_user(ee, uclsp.reshape(-1), relation_emb,
                             (icol2, irow2, ival),
                             (ccol2, cu2, cuc, cval))
        ee, ent_res = _finish(ent_parts, ent_res, 1000)
        ue, usr_res = _finish(usr_parts, usr_res, 1024)

    cor = _cor(disen_weight_att)
    return ent_res, usr_res, cor


# speedup vs baseline: 1.9719x; 1.3482x over previous
import functools

import jax
import jax.numpy as jnp
from jax import lax
from jax.experimental import pallas as pl
from jax.experimental.pallas import tpu as pltpu
from jax.experimental.pallas import tpu_sc as plsc

N_ENT = 10000
N_USERS = 4096
N_ITEMS = 5000
N_REL = 24
N_FACTORS = 4
N_CLUSTERS = 4
D = 128
N_EDGES = 320000
NNZ = 200000
N_HOPS = 2

NC = 2
NS = 16
NW = NC * NS
SB = 128

E_PAD = 327680
EPT = E_PAD // NW
NNZ_PAD = 212992
ZPT = NNZ_PAD // NW

ENT_STRIPE = 632
ENT_LAST = N_ENT - 15 * ENT_STRIPE
USR_STRIPE = N_USERS // NS


def _pad1(x, n, val):
    return jnp.pad(x, (0, n - x.shape[0]), constant_values=val)


def _softmax_body(ncols, x_ref, w_ref, o_ref):
    z = lax.dot_general(x_ref[...], w_ref[...], (((1,), (1,)), ((), ())),
                        preferred_element_type=jnp.float32)
    col = lax.broadcasted_iota(jnp.int32, z.shape, 1)
    z = jnp.where(col < ncols, z, -jnp.inf)
    z = z - jnp.max(z, axis=1, keepdims=True)
    e = jnp.exp(z)
    o_ref[...] = e / jnp.sum(e, axis=1, keepdims=True)


def _prep_softmax(x, w_pad, ncols, blk):
    n, _ = x.shape
    wp = w_pad.shape[0]
    return pl.pallas_call(
        functools.partial(_softmax_body, ncols),
        grid=(n // blk,),
        in_specs=[pl.BlockSpec((blk, D), lambda i: (i, 0)),
                  pl.BlockSpec((wp, D), lambda i: (0, 0))],
        out_specs=pl.BlockSpec((blk, wp), lambda i: (i, 0)),
        out_shape=jax.ShapeDtypeStruct((n, wp), jnp.float32),
    )(x, w_pad)


def _finish_body(parts_ref, res_ref, e_ref, r_ref):
    x = parts_ref[0] + parts_ref[1]
    nrm = jnp.sqrt(jnp.sum(x * x, axis=1, keepdims=True))
    y = x / jnp.maximum(nrm, 1e-12)
    e_ref[...] = y
    r_ref[...] = res_ref[...] + y


def _finish(parts, res, blk):
    n = res.shape[0]
    return pl.pallas_call(
        _finish_body,
        grid=(n // blk,),
        in_specs=[pl.BlockSpec((2, blk, D), lambda i: (0, i, 0)),
                  pl.BlockSpec((blk, D), lambda i: (i, 0))],
        out_specs=[pl.BlockSpec((blk, D), lambda i: (i, 0)),
                   pl.BlockSpec((blk, D), lambda i: (i, 0))],
        out_shape=[jax.ShapeDtypeStruct((n, D), jnp.float32),
                   jax.ShapeDtypeStruct((n, D), jnp.float32)],
    )(parts, res)


def _cor_body(w_ref, o_ref):
    f = N_FACTORS
    n = N_REL
    ri = lax.broadcasted_iota(jnp.int32, (n, n), 0)
    ci = lax.broadcasted_iota(jnp.int32, (n, n), 1)

    def centered_dist(row):
        x = jnp.broadcast_to(row, (n, n))
        dg = jnp.where(ri == ci, x, 0.0)
        vcol = jnp.sum(dg, axis=1, keepdims=True)
        outer2 = (vcol * row) * 2.0
        sq = vcol * vcol
        sqr = row * row
        a = jnp.sqrt(jnp.maximum(sq - outer2 + sqr, 0.0) + 1e-08)
        return (a - jnp.mean(a, axis=0, keepdims=True)
                - jnp.mean(a, axis=1, keepdims=True) + jnp.mean(a))

    mats = [centered_dist(w_ref[i:i + 1, :]) for i in range(f)]
    n2 = float(n * n)
    cor = jnp.float32(0.0)
    for i in range(f):
        for j in range(i + 1, f):
            ab = jnp.sqrt(jnp.maximum(jnp.sum(mats[i] * mats[j]) / n2, 0.0) + 1e-08)
            aa = jnp.sqrt(jnp.maximum(jnp.sum(mats[i] * mats[i]) / n2, 0.0) + 1e-08)
            bb = jnp.sqrt(jnp.maximum(jnp.sum(mats[j] * mats[j]) / n2, 0.0) + 1e-08)
            cor = cor + ab / jnp.sqrt(aa * bb + 1e-08)
    o_ref[...] = jnp.broadcast_to(cor, (1, 1))


def _cor(disen_weight_att):
    out = pl.pallas_call(
        _cor_body,
        out_shape=jax.ShapeDtypeStruct((1, 1), jnp.float32),
    )(disen_weight_att)
    return out[0, 0]


KE = 128
NSB_E = KE // SB
NCH_E = EPT // KE
KU = 256
NSB_U = KU // SB
NCH_U = ZPT // KU


def _zero_buf(buf, k):
    def _row(r, _):
        for c in range(8):
            buf[r, pl.ds(c * 16, 16)] = jnp.zeros((16,), jnp.float32)
        return _
    lax.fori_loop(0, k, _row, None)


def _zero_stripe(buf, k, acc, off, rows):
    done = 0
    while done < rows:
        n = min(k, rows - done)
        pltpu.sync_copy(buf.at[pl.ds(0, n)], acc.at[pl.ds(off + done, n)])
        done += n


def _ring_phase(wid, per_tile, nch, k, nsb, acc,
                ee_t, idx_b, sem_i, sem_g, sem_s,
                issue_idx, issue_gather, compute):
    def pair(p, _):
        base0 = wid * per_tile + p * (2 * k)
        hi0 = issue_idx(base0, 0, sem_i)
        hi1 = issue_idx(base0 + k, 1, sem_i)
        for h in hi0:
            h.wait()
        hg0 = issue_gather(0, sem_g)
        for h in hi1:
            h.wait()
        hg1 = issue_gather(1, sem_g)
        for h in hg0:
            h.wait()
        compute(0)
        hs0 = [pltpu.async_copy(ee_t.at[pl.ds(j * SB, SB)],
                                acc.at[idx_b.at[j]], sem_s, add=True)
               for j in range(nsb)]
        for h in hg1:
            h.wait()
        compute(1)
        hs1 = [pltpu.async_copy(ee_t.at[pl.ds(k + j * SB, SB)],
                                acc.at[idx_b.at[nsb + j]], sem_s, add=True)
               for j in range(nsb)]
        for h in hs0 + hs1:
            h.wait()
        return _
    lax.fori_loop(0, nch // 2, pair, None)


def _sc_edge_body(ee, attp, rel,
                  tail1, head1, etyp, eimp, eatt,
                  out_e,
                  ee_t, asel_v, rel_v, typ_v, w_v, idx_a, idx_b, idx_c,
                  sem_i, sem_g, sem_s,
                  ent_acc):
    cid = lax.axis_index("c")
    sid = lax.axis_index("s")
    wid = sid * NC + cid

    pltpu.sync_copy(rel, rel_v)
    _zero_buf(ee_t, 2 * KE)

    @pl.when(sid < 15)
    def _():
        _zero_stripe(ee_t, 2 * KE, ent_acc, sid * ENT_STRIPE, ENT_STRIPE)

    @pl.when(sid == 15)
    def _():
        _zero_stripe(ee_t, 2 * KE, ent_acc, 15 * ENT_STRIPE, ENT_LAST)

    plsc.subcore_barrier()

    def issue_idx(base, b, sem):
        hs = [pltpu.async_copy(etyp.at[pl.ds(base, KE)], typ_v.at[b], sem),
              pltpu.async_copy(eimp.at[pl.ds(base, KE)], w_v.at[b], sem)]
        for j in range(NSB_E):
            o = base + j * SB
            r = b * NSB_E + j
            hs.append(pltpu.async_copy(tail1.at[pl.ds(o, SB)], idx_a.at[r], sem))
            hs.append(pltpu.async_copy(head1.at[pl.ds(o, SB)], idx_b.at[r], sem))
            hs.append(pltpu.async_copy(eatt.at[pl.ds(o, SB)], idx_c.at[r], sem))
        return hs

    def issue_gather(b, sem):
        hs = []
        for j in range(NSB_E):
            r = b * NSB_E + j
            hs.append(pltpu.async_copy(
                ee.at[idx_a.at[r]], ee_t.at[pl.ds(b * KE + j * SB, SB)], sem))
            hs.append(pltpu.async_copy(
                attp.at[idx_c.at[r]], asel_v.at[b, pl.ds(j * SB, SB)], sem))
        return hs

    def compute(b):
        def group(g, _):
            t16 = typ_v[b, pl.ds(g * 16, 16)]
            wful = asel_v[b, pl.ds(g * 16, 16)] * w_v[b, pl.ds(g * 16, 16)]
            for j in range(16):
                w = wful[j]
                t = t16[j]
                e = b * KE + g * 16 + j
                for c in range(8):
                    sl = pl.ds(c * 16, 16)
                    ee_t[e, sl] = ee_t[e, sl] * rel_v[t, sl] * w
            return _
        lax.fori_loop(0, KE // 16, group, None)

    _ring_phase(wid, EPT, NCH_E, KE, NSB_E, ent_acc,
                ee_t, idx_b, sem_i, sem_g, sem_s,
                issue_idx, issue_gather, compute)

    plsc.subcore_barrier()

    @pl.when(sid < 15)
    def _():
        pltpu.sync_copy(ent_acc.at[pl.ds(sid * ENT_STRIPE, ENT_STRIPE)],
                        out_e.at[cid, pl.ds(sid * ENT_STRIPE, ENT_STRIPE)])

    @pl.when(sid == 15)
    def _():
        pltpu.sync_copy(ent_acc.at[pl.ds(15 * ENT_STRIPE, ENT_LAST)],
                        out_e.at[cid, pl.ds(15 * ENT_STRIPE, ENT_LAST)])


def _sc_user_body(ee, uclsp, rel,
                  icol1, irow1, ival,
                  ccol1, cu1, cuc, cval,
                  out_u,
                  ee_t, asel_v, rel_v, rs_v, w_v, idx_a, idx_b, idx_c,
                  sem_i, sem_g, sem_s,
                  usr_acc):
    cid = lax.axis_index("c")
    sid = lax.axis_index("s")
    wid = sid * NC + cid

    pltpu.sync_copy(rel, rel_v)
    for c in range(8):
        sl = pl.ds(c * 16, 16)
        acc = jnp.zeros((16,), jnp.float32)
        for r in range(N_REL):
            acc = acc + rel_v[r, sl]
        rs_v[sl] = acc

    _zero_buf(ee_t, 2 * KU)
    _zero_stripe(ee_t, 2 * KU, usr_acc, sid * USR_STRIPE, USR_STRIPE)
    plsc.subcore_barrier()

    def i_issue_idx(base, b, sem):
        hs = [pltpu.async_copy(ival.at[pl.ds(base, KU)], w_v.at[b], sem)]
        for j in range(NSB_U):
            o = base + j * SB
            r = b * NSB_U + j
            hs.append(pltpu.async_copy(icol1.at[pl.ds(o, SB)], idx_a.at[r], sem))
            hs.append(pltpu.async_copy(irow1.at[pl.ds(o, SB)], idx_b.at[r], sem))
        return hs

    def i_issue_gather(b, sem):
        return [pltpu.async_copy(ee.at[idx_a.at[b * NSB_U + j]],
                                 ee_t.at[pl.ds(b * KU + j * SB, SB)], sem)
                for j in range(NSB_U)]

    def i_compute(b):
        def group(g, _):
            wful = w_v[b, pl.ds(g * 16, 16)]
            for j in range(16):
                w = wful[j]
                e = b * KU + g * 16 + j
                for c in range(8):
                    sl = pl.ds(c * 16, 16)
                    ee_t[e, sl] = ee_t[e, sl] * w
            return _
        lax.fori_loop(0, KU // 16, group, None)

    _ring_phase(wid, ZPT, NCH_U, KU, NSB_U, usr_acc,
                ee_t, idx_b, sem_i, sem_g, sem_s,
                i_issue_idx, i_issue_gather, i_compute)

    def c_issue_idx(base, b, sem):
        hs = [pltpu.async_copy(cval.at[pl.ds(base, KU)], w_v.at[b], sem)]
        for j in range(NSB_U):
            o = base + j * SB
            r = b * NSB_U + j
            hs.append(pltpu.async_copy(ccol1.at[pl.ds(o, SB)], idx_a.at[r], sem))
            hs.append(pltpu.async_copy(cu1.at[pl.ds(o, SB)], idx_b.at[r], sem))
            hs.append(pltpu.async_copy(cuc.at[pl.ds(o, SB)], idx_c.at[r], sem))
        return hs

    def c_issue_gather(b, sem):
        hs = []
        for j in range(NSB_U):
            r = b * NSB_U + j
            hs.append(pltpu.async_copy(
                ee.at[idx_a.at[r]], ee_t.at[pl.ds(b * KU + j * SB, SB)], sem))
            hs.append(pltpu.async_copy(
                uclsp.at[idx_c.at[r]], asel_v.at[b, pl.ds(j * SB, SB)], sem))
        return hs

    def c_compute(b):
        def group(g, _):
            wful = asel_v[b, pl.ds(g * 16, 16)] * w_v[b, pl.ds(g * 16, 16)]
            for j in range(16):
                w = wful[j]
                e = b * KU + g * 16 + j
                for c in range(8):
                    sl = pl.ds(c * 16, 16)
                    ee_t[e, sl] = ee_t[e, sl] * rs_v[sl] * w
            return _
        lax.fori_loop(0, KU // 16, group, None)

    _ring_phase(wid, ZPT, NCH_U, KU, NSB_U, usr_acc,
                ee_t, idx_b, sem_i, sem_g, sem_s,
                c_issue_idx, c_issue_gather, c_compute)

    plsc.subcore_barrier()
    pltpu.sync_copy(usr_acc.at[pl.ds(sid * USR_STRIPE, USR_STRIPE)],
                    out_u.at[cid, pl.ds(sid * USR_STRIPE, USR_STRIPE)])


def _sc_edge(ee, attp, rel, edge_in):
    tail1, head1, etyp, eimp, eatt = edge_in
    mesh = plsc.VectorSubcoreMesh(core_axis_name="c", subcore_axis_name="s")
    f32 = jnp.float32
    kfn = pl.kernel(
        _sc_edge_body,
        mesh=mesh,
        out_type=jax.ShapeDtypeStruct((NC, N_ENT, D), f32),
        scratch_types=[
            pltpu.VMEM((2 * KE, D), f32),
            pltpu.VMEM((2, KE), f32),
            pltpu.VMEM((N_REL, D), f32),
            pltpu.VMEM((2, KE), jnp.int32),
            pltpu.VMEM((2, KE), f32),
            pltpu.VMEM((2 * NSB_E, SB), jnp.int32),
            pltpu.VMEM((2 * NSB_E, SB), jnp.int32),
            pltpu.VMEM((2 * NSB_E, SB), jnp.int32),
            pltpu.SemaphoreType.DMA,
            pltpu.SemaphoreType.DMA,
            pltpu.SemaphoreType.DMA,
            pltpu.VMEM_SHARED((N_ENT, D), f32),
        ],
    )
    return kfn(ee, attp, rel, tail1, head1, etyp, eimp, eatt)


def _sc_user(ee, uclsp, rel, inter_in, cls_in):
    icol1, irow1, ival = inter_in
    ccol1, cu1, cuc, cval = cls_in
    mesh = plsc.VectorSubcoreMesh(core_axis_name="c", subcore_axis_name="s")
    f32 = jnp.float32
    kfn = pl.kernel(
        _sc_user_body,
        mesh=mesh,
        out_type=jax.ShapeDtypeStruct((NC, N_USERS, D), f32),
        scratch_types=[
            pltpu.VMEM((2 * KU, D), f32),
            pltpu.VMEM((2, KU), f32),
            pltpu.VMEM((N_REL, D), f32),
            pltpu.VMEM((D,), f32),
            pltpu.VMEM((2, KU), f32),
            pltpu.VMEM((2 * NSB_U, SB), jnp.int32),
            pltpu.VMEM((2 * NSB_U, SB), jnp.int32),
            pltpu.VMEM((2 * NSB_U, SB), jnp.int32),
            pltpu.SemaphoreType.DMA,
            pltpu.SemaphoreType.DMA,
            pltpu.SemaphoreType.DMA,
            pltpu.VMEM_SHARED((N_USERS, D), f32),
        ],
    )
    return kfn(ee, uclsp, rel, icol1, irow1, ival, ccol1, cu1, cuc, cval)


def kernel(user_emb, entity_emb, latent_emb, relation_emb, disen_weight_att,
           ent_rel_w, usr_cls_w, edge_index, edge_type, edge_imp,
           inter_rows, inter_cols, inter_vals, cls_rows, cls_cols, cls_vals):
    del latent_emb, ent_rel_w
    i32 = jnp.int32
    head = edge_index[0].astype(i32)
    tail = edge_index[1].astype(i32)

    tail2 = _pad1(tail, E_PAD, 0)
    head2 = _pad1(head, E_PAD, 0)
    etyp = _pad1(edge_type.astype(i32), E_PAD, 0)
    eimp = _pad1(edge_imp, E_PAD, 0.0)

    icol2 = _pad1(inter_cols.astype(i32), NNZ_PAD, 0)
    irow2 = _pad1(inter_rows.astype(i32), NNZ_PAD, 0)
    ival = _pad1(inter_vals, NNZ_PAD, 0.0)

    cr = cls_rows.astype(i32)
    ccol2 = _pad1(cls_cols.astype(i32), NNZ_PAD, 0)
    cu2 = _pad1(cr % N_USERS, NNZ_PAD, 0)
    cuc = _pad1((cr % N_USERS) * 16 + cr // N_USERS, NNZ_PAD, 0)
    cval = _pad1(cls_vals, NNZ_PAD, 0.0)
    eatt = _pad1(head * 32 + edge_type.astype(i32), E_PAD, 0)

    rel_p = jnp.pad(relation_emb, ((0, 32 - N_REL), (0, 0)))
    uclsw_p = jnp.pad(usr_cls_w, ((0, 16 - N_CLUSTERS), (0, 0)))

    ee = entity_emb
    ue = user_emb
    ent_res = entity_emb
    usr_res = user_emb
    for _ in range(N_HOPS):
        attp = _prep_softmax(ee, rel_p, N_REL, 1000)
        uclsp = _prep_softmax(ue, uclsw_p, N_CLUSTERS, 1024)
        ent_parts = _sc_edge(ee, attp.reshape(-1), relation_emb,
                             (tail2, head2, etyp, eimp, eatt))
        usr_parts = _sc_user(ee, uclsp.reshape(-1), relation_emb,
                             (icol2, irow2, ival),
                             (ccol2, cu2, cuc, cval))
        ee, ent_res = _finish(ent_parts, ent_res, 1000)
        ue, usr_res = _finish(usr_parts, usr_res, 1024)

    cor = _cor(disen_weight_att)
    return ent_res, usr_res, cor
